# Initial kernel scaffold; baseline (speedup 1.0000x reference)
import functools

import jax
import jax.numpy as jnp
from jax import lax
from jax.experimental import pallas as pl
from jax.experimental.pallas import tpu as pltpu
from jax.experimental.pallas import tpu_sc as plsc

BATCH = 16384
EMB_DIM = 512
L = 16
NC, NS = 2, 16
NW = NC * NS
BPW = BATCH // NW
CHUNK = 16
NCHUNK = BPW // CHUNK
DCHUNK = EMB_DIM // L


def _sc_kernel(u_hbm, i_hbm, j_hbm, ue_hbm, al_hbm, ii_hbm, it_hbm, ik_hbm,
               out_hbm,
               idx_u, idx_i, idx_j,
               u_rows, a_rows, ii_r, it_r, ik_r, ji_r, jt_r, jk_r,
               out_v, sem):
    wid = lax.axis_index("s") * NC + lax.axis_index("c")
    base = wid * BPW
    pltpu.sync_copy(u_hbm.at[pl.ds(base, BPW)], idx_u)
    pltpu.sync_copy(i_hbm.at[pl.ds(base, BPW)], idx_i)
    pltpu.sync_copy(j_hbm.at[pl.ds(base, BPW)], idx_j)

    lane = lax.iota(jnp.int32, L)
    mask3 = lane < 3

    @pl.loop(0, NCHUNK)
    def _chunk(c):
        off = c * CHUNK
        iu = idx_u.at[pl.ds(off, CHUNK)]
        ii_ = idx_i.at[pl.ds(off, CHUNK)]
        ij = idx_j.at[pl.ds(off, CHUNK)]
        copies = [
            pltpu.async_copy(ue_hbm.at[iu], u_rows, sem),
            pltpu.async_copy(al_hbm.at[iu], a_rows, sem),
            pltpu.async_copy(ii_hbm.at[ii_], ii_r, sem),
            pltpu.async_copy(it_hbm.at[ii_], it_r, sem),
            pltpu.async_copy(ik_hbm.at[ii_], ik_r, sem),
            pltpu.async_copy(ii_hbm.at[ij], ji_r, sem),
            pltpu.async_copy(it_hbm.at[ij], jt_r, sem),
            pltpu.async_copy(ik_hbm.at[ij], jk_r, sem),
        ]
        for cp in copies:
            cp.wait()

        def elem_body(b, res_vec):
            def dim_body(d, accs):
                a_im, a_tx, a_kg = accs
                sl = pl.ds(d * L, L)
                uv = u_rows[b, sl]
                a_im = a_im + uv * (ii_r[b, sl] - ji_r[b, sl])
                a_tx = a_tx + uv * (it_r[b, sl] - jt_r[b, sl])
                a_kg = a_kg + uv * (ik_r[b, sl] - jk_r[b, sl])
                return a_im, a_tx, a_kg

            zero = jnp.zeros((L,), jnp.float32)
            a_im, a_tx, a_kg = lax.fori_loop(0, DCHUNK, dim_body,
                                             (zero, zero, zero))
            d_im = jnp.sum(a_im)
            d_tx = jnp.sum(a_tx)
            d_kg = jnp.sum(a_kg)

            av = a_rows[b, :]
            m = jnp.max(jnp.where(mask3, av, -1e30))
            e = jnp.where(mask3, jnp.exp(av - m), 0.0)
            s = jnp.sum(e)
            dvec = jnp.where(lane == 0, d_im,
                             jnp.where(lane == 1, d_tx,
                                       jnp.where(lane == 2, d_kg, 0.0)))
            res = jnp.sum(e * dvec) / s
            return jnp.where(lane == b, res, res_vec)

        res_vec = lax.fori_loop(0, CHUNK, elem_body,
                                jnp.zeros((L,), jnp.float32))
        out_v[pl.ds(off, CHUNK)] = res_vec

    pltpu.sync_copy(out_v, out_hbm.at[pl.ds(base, BPW)])


def kernel(u, i, j, user_emb, alpha_emb, item_embed_img, item_embed_txt,
           item_embed_kg):
    alpha16 = jnp.pad(alpha_emb, ((0, 0), (0, L - 3)))
    mesh = plsc.VectorSubcoreMesh(core_axis_name="c", subcore_axis_name="s")

    run = functools.partial(
        pl.kernel,
        out_type=jax.ShapeDtypeStruct((BATCH,), jnp.float32),
        mesh=mesh,
        scratch_types=[
            pltpu.VMEM((BPW,), jnp.int32),
            pltpu.VMEM((BPW,), jnp.int32),
            pltpu.VMEM((BPW,), jnp.int32),
            pltpu.VMEM((CHUNK, EMB_DIM), jnp.float32),
            pltpu.VMEM((CHUNK, L), jnp.float32),
            pltpu.VMEM((CHUNK, EMB_DIM), jnp.float32),
            pltpu.VMEM((CHUNK, EMB_DIM), jnp.float32),
            pltpu.VMEM((CHUNK, EMB_DIM), jnp.float32),
            pltpu.VMEM((CHUNK, EMB_DIM), jnp.float32),
            pltpu.VMEM((CHUNK, EMB_DIM), jnp.float32),
            pltpu.VMEM((CHUNK, EMB_DIM), jnp.float32),
            pltpu.VMEM((BPW,), jnp.float32),
            pltpu.SemaphoreType.DMA,
        ],
    )(_sc_kernel)
    return run(u.astype(jnp.int32), i.astype(jnp.int32), j.astype(jnp.int32),
               user_emb, alpha16, item_embed_img, item_embed_txt,
               item_embed_kg)


# speedup vs baseline: 1.3564x; 1.3564x over previous
import dataclasses
import functools

import jax
import jax.numpy as jnp
from jax import lax
from jax.experimental import pallas as pl
from jax.experimental.pallas import tpu as pltpu
from jax.experimental.pallas import tpu_sc as plsc

BATCH = 16384
EMB_DIM = 512
L = 16
NC, NS = 2, 16
NW = NC * NS
BPW = BATCH // NW
CHUNK = 16
NCHUNK = BPW // CHUNK
DCHUNK = EMB_DIM // L


def _sc_kernel(u_hbm, i_hbm, j_hbm, ue_hbm, al_hbm, ii_hbm, it_hbm, ik_hbm,
               out_hbm,
               idx_u, idx_i, idx_j,
               u_rows, a_rows, ii_r, it_r, ik_r, ji_r, jt_r, jk_r,
               out_v, sem):
    wid = lax.axis_index("s") * NC + lax.axis_index("c")
    base = wid * BPW
    pltpu.sync_copy(u_hbm.at[pl.ds(base, BPW)], idx_u)
    pltpu.sync_copy(i_hbm.at[pl.ds(base, BPW)], idx_i)
    pltpu.sync_copy(j_hbm.at[pl.ds(base, BPW)], idx_j)

    lane = lax.iota(jnp.int32, L)
    mask3 = lane < 3

    @pl.loop(0, NCHUNK)
    def _chunk(c):
        off = c * CHUNK
        iu = idx_u.at[pl.ds(off, CHUNK)]
        ii_ = idx_i.at[pl.ds(off, CHUNK)]
        ij = idx_j.at[pl.ds(off, CHUNK)]
        copies = [
            pltpu.async_copy(ue_hbm.at[iu], u_rows, sem),
            pltpu.async_copy(al_hbm.at[iu], a_rows, sem),
            pltpu.async_copy(ii_hbm.at[ii_], ii_r, sem),
            pltpu.async_copy(it_hbm.at[ii_], it_r, sem),
            pltpu.async_copy(ik_hbm.at[ii_], ik_r, sem),
            pltpu.async_copy(ii_hbm.at[ij], ji_r, sem),
            pltpu.async_copy(it_hbm.at[ij], jt_r, sem),
            pltpu.async_copy(ik_hbm.at[ij], jk_r, sem),
        ]
        for cp in copies:
            cp.wait()

        def elem_body(b, res_vec):
            def dim_body(d, accs):
                a_im, a_tx, a_kg = accs
                sl = pl.ds(d * L, L)
                uv = u_rows[b, sl]
                a_im = a_im + uv * (ii_r[b, sl] - ji_r[b, sl])
                a_tx = a_tx + uv * (it_r[b, sl] - jt_r[b, sl])
                a_kg = a_kg + uv * (ik_r[b, sl] - jk_r[b, sl])
                return a_im, a_tx, a_kg

            zero = jnp.zeros((L,), jnp.float32)
            a_im, a_tx, a_kg = lax.fori_loop(0, DCHUNK, dim_body,
                                             (zero, zero, zero))
            d_im = jnp.sum(a_im)
            d_tx = jnp.sum(a_tx)
            d_kg = jnp.sum(a_kg)

            av = a_rows[b, pl.ds(0, L)]
            m = jnp.max(jnp.where(mask3, av, -1e30))
            e = jnp.where(mask3, jnp.exp(av - m), 0.0)
            s = jnp.sum(e)
            dvec = jnp.where(lane == 0, d_im,
                             jnp.where(lane == 1, d_tx,
                                       jnp.where(lane == 2, d_kg, 0.0)))
            res = jnp.sum((e * dvec) / s)
            return jnp.where(lane == b, res, res_vec)

        res_vec = lax.fori_loop(0, CHUNK, elem_body,
                                jnp.zeros((L,), jnp.float32))
        out_v[pl.ds(off, CHUNK)] = res_vec

    pltpu.sync_copy(out_v, out_hbm.at[pl.ds(base, BPW)])


def kernel(u, i, j, user_emb, alpha_emb, item_embed_img, item_embed_txt,
           item_embed_kg):
    alpha16 = jnp.pad(alpha_emb, ((0, 0), (0, 128 - 3)))
    mesh = plsc.VectorSubcoreMesh(core_axis_name="c", subcore_axis_name="s")

    cp = pltpu.CompilerParams()
    if "needs_layout_passes" in pltpu.CompilerParams.__dataclass_fields__:
        cp = dataclasses.replace(cp, needs_layout_passes=False)

    run = functools.partial(
        pl.kernel,
        out_type=jax.ShapeDtypeStruct((BATCH,), jnp.float32),
        mesh=mesh,
        compiler_params=cp,
        scratch_types=[
            pltpu.VMEM((BPW,), jnp.int32),
            pltpu.VMEM((BPW,), jnp.int32),
            pltpu.VMEM((BPW,), jnp.int32),
            pltpu.VMEM((CHUNK, EMB_DIM), jnp.float32),
            pltpu.VMEM((CHUNK, 128), jnp.float32),
            pltpu.VMEM((CHUNK, EMB_DIM), jnp.float32),
            pltpu.VMEM((CHUNK, EMB_DIM), jnp.float32),
            pltpu.VMEM((CHUNK, EMB_DIM), jnp.float32),
            pltpu.VMEM((CHUNK, EMB_DIM), jnp.float32),
            pltpu.VMEM((CHUNK, EMB_DIM), jnp.float32),
            pltpu.VMEM((CHUNK, EMB_DIM), jnp.float32),
            pltpu.VMEM((BPW,), jnp.float32),
            pltpu.SemaphoreType.DMA,
        ],
    )(_sc_kernel)
    return run(u.astype(jnp.int32), i.astype(jnp.int32), j.astype(jnp.int32),
               user_emb, alpha16, item_embed_img, item_embed_txt,
               item_embed_kg)
